# Initial kernel scaffold; baseline (speedup 1.0000x reference)
import jax
import jax.numpy as jnp
from jax.experimental import pallas as pl


def kernel(x, edge_index, batch, W1, b1, W2, b2):
    raise NotImplementedError("write your pallas kernel here")


# speedup vs baseline: 28.6129x; 28.6129x over previous
import functools

import jax
import jax.numpy as jnp
from jax import lax
from jax.experimental import pallas as pl
from jax.experimental.pallas import tpu as pltpu
from jax.experimental.pallas import tpu_sc as plsc

N = 10000
E = 320000
G = 64
IN_DIM = 128
HID = 32
OUT = 32

NC = 2
NS = 16
NW = NC * NS
C = 128
K = 79
EP_TILE = C * K
E_PAD = EP_TILE * NW
STRIPE = 632
N_ACC = STRIPE * NS
PAD_DST = N
RB = 1000
GRID = N // RB

_SC_PARAMS = pltpu.CompilerParams(use_tc_tiling_on_sc=False)


def _zero_vmem(ref, nrows, ncols):
    z = jnp.zeros((16,), jnp.float32)

    def body(i, _):
        for j in range(ncols // 16):
            ref[i, pl.ds(j * 16, 16)] = z
        return 0

    lax.fori_loop(0, nrows, body, 0)


def _zero_stripe(buf_v, acc_sh, s, ncols):
    _zero_vmem(buf_v, C, ncols)
    base = s * STRIPE
    for off in range(0, STRIPE - C + 1, C):
        pltpu.sync_copy(buf_v, acc_sh.at[pl.ds(base + off, C)])
    rem = STRIPE % C
    if rem:
        pltpu.sync_copy(
            buf_v.at[pl.ds(0, rem)],
            acc_sh.at[pl.ds(base + STRIPE - rem, rem)],
        )


@functools.cache
def _sc_kernels():
    mesh = plsc.VectorSubcoreMesh(core_axis_name="c", subcore_axis_name="s")

    @functools.partial(
        pl.kernel,
        out_type=jax.ShapeDtypeStruct((NC, N_ACC, 16), jnp.float32),
        mesh=mesh,
        compiler_params=_SC_PARAMS,
        scratch_types=[
            pltpu.VMEM((K, C), jnp.int32),
            pltpu.VMEM((C, 16), jnp.float32),
            pltpu.VMEM_SHARED((N_ACC, 16), jnp.float32),
        ],
    )
    def _sc_degree(dst_hbm, out_hbm, idx_v, ones_v, acc_sh):
        c = lax.axis_index("c")
        s = lax.axis_index("s")
        wid = s * NC + c

        _zero_stripe(ones_v, acc_sh, s, 16)

        one = jnp.ones((16,), jnp.float32)

        def fill(i, _):
            ones_v[i, pl.ds(0, 16)] = one
            return 0

        lax.fori_loop(0, C, fill, 0)

        pltpu.sync_copy(dst_hbm.at[wid], idx_v)
        plsc.subcore_barrier()

        def body(k, _):
            pltpu.sync_copy(ones_v, acc_sh.at[idx_v.at[k]], add=True)
            return 0

        lax.fori_loop(0, K, body, 0)
        plsc.subcore_barrier()
        pltpu.sync_copy(
            acc_sh.at[pl.ds(s * STRIPE, STRIPE)],
            out_hbm.at[c, pl.ds(s * STRIPE, STRIPE)],
        )

    @functools.partial(
        pl.kernel,
        out_type=jax.ShapeDtypeStruct((NC, N_ACC, HID), jnp.float32),
        mesh=mesh,
        compiler_params=_SC_PARAMS,
        scratch_types=[
            pltpu.VMEM((K, C), jnp.int32),
            pltpu.VMEM((K, C), jnp.int32),
            pltpu.VMEM((C, HID), jnp.float32),
            pltpu.VMEM_SHARED((N_ACC, HID), jnp.float32),
            pltpu.SemaphoreType.DMA,
        ],
    )
    def _sc_scatter(h_hbm, src_hbm, dst_hbm, out_hbm, src_v, dst_v, rows_v,
                    acc_sh, sem):
        c = lax.axis_index("c")
        s = lax.axis_index("s")
        wid = s * NC + c

        _zero_stripe(rows_v, acc_sh, s, HID)

        pltpu.sync_copy(src_hbm.at[wid], src_v)
        pltpu.sync_copy(dst_hbm.at[wid], dst_v)
        plsc.subcore_barrier()

        def body(k, _):
            pltpu.async_copy(h_hbm.at[src_v.at[k]], rows_v, sem).wait()
            pltpu.sync_copy(rows_v, acc_sh.at[dst_v.at[k]], add=True)
            return 0

        lax.fori_loop(0, K, body, 0)
        plsc.subcore_barrier()
        pltpu.sync_copy(
            acc_sh.at[pl.ds(s * STRIPE, STRIPE)],
            out_hbm.at[c, pl.ds(s * STRIPE, STRIPE)],
        )

    return _sc_degree, _sc_scatter


def _dinv_from(degp):
    deg = degp[0] + degp[1] + 1.0
    return lax.rsqrt(deg[:, 0:1])


def _tc_mm1_body(x_ref, w_ref, degp_ref, out_ref):
    dinv = _dinv_from(degp_ref[...])
    h = jnp.dot(x_ref[...], w_ref[...], preferred_element_type=jnp.float32)
    out_ref[...] = h * dinv


def _tc_mid_body(h_ref, sp_ref, degp_ref, w_ref, b_ref, out_ref):
    dinv = _dinv_from(degp_ref[...])
    sp = sp_ref[...]
    z = (h_ref[...] + sp[0] + sp[1]) * dinv + b_ref[...]
    h = jnp.maximum(z, 0.0)
    out_ref[...] = (
        jnp.dot(h, w_ref[...], preferred_element_type=jnp.float32) * dinv
    )


def _tc_pool_body(h_ref, qp_ref, degp_ref, b_ref, batch_ref, out_ref,
                  acc_ref, cnt_ref):
    i = pl.program_id(0)

    @pl.when(i == 0)
    def _():
        acc_ref[...] = jnp.zeros((G, OUT), jnp.float32)
        cnt_ref[...] = jnp.zeros((G, OUT), jnp.float32)

    dinv = _dinv_from(degp_ref[...])
    qp = qp_ref[...]
    rows = (h_ref[...] + qp[0] + qp[1]) * dinv + b_ref[...]
    gid = lax.broadcasted_iota(jnp.int32, (G, RB), 0)
    onehot = (gid == jnp.broadcast_to(batch_ref[0], (G, RB))).astype(
        jnp.float32)
    acc_ref[...] += jnp.dot(onehot, rows, preferred_element_type=jnp.float32)
    cnt_ref[...] += jnp.broadcast_to(
        jnp.sum(onehot, axis=1, keepdims=True), (G, OUT))

    @pl.when(i == GRID - 1)
    def _():
        out_ref[...] = acc_ref[...] / jnp.maximum(cnt_ref[...], 1.0)


_row_spec = lambda w: pl.BlockSpec((RB, w), lambda i: (i, 0))
_part_spec = lambda w: pl.BlockSpec((NC, RB, w), lambda i: (0, i, 0))
_full_spec = lambda a, b: pl.BlockSpec((a, b), lambda i: (0, 0))

_tc_mm1 = pl.pallas_call(
    _tc_mm1_body,
    grid=(GRID,),
    in_specs=[_row_spec(IN_DIM), _full_spec(IN_DIM, HID), _part_spec(16)],
    out_specs=_row_spec(HID),
    out_shape=jax.ShapeDtypeStruct((N, HID), jnp.float32),
)

_tc_mid = pl.pallas_call(
    _tc_mid_body,
    grid=(GRID,),
    in_specs=[_row_spec(HID), _part_spec(HID), _part_spec(16),
              _full_spec(HID, OUT), _full_spec(1, HID)],
    out_specs=_row_spec(OUT),
    out_shape=jax.ShapeDtypeStruct((N, OUT), jnp.float32),
)

_tc_pool = pl.pallas_call(
    _tc_pool_body,
    grid=(GRID,),
    in_specs=[_row_spec(OUT), _part_spec(OUT), _part_spec(16),
              _full_spec(1, OUT),
              pl.BlockSpec((1, 1, RB), lambda i: (i, 0, 0))],
    out_specs=_full_spec(G, OUT),
    out_shape=jax.ShapeDtypeStruct((G, OUT), jnp.float32),
    scratch_shapes=[pltpu.VMEM((G, OUT), jnp.float32),
                    pltpu.VMEM((G, OUT), jnp.float32)],
)


def kernel(x, edge_index, batch, W1, b1, W2, b2):
    src = edge_index[0].astype(jnp.int32)
    dst = edge_index[1].astype(jnp.int32)
    pad = E_PAD - E
    src3d = jnp.concatenate(
        [src, jnp.zeros((pad,), jnp.int32)]).reshape(NW, K, C)
    dst3d = jnp.concatenate(
        [dst, jnp.full((pad,), PAD_DST, jnp.int32)]).reshape(NW, K, C)
    batch3d = batch.astype(jnp.int32).reshape(GRID, 1, RB)

    sc_degree, sc_scatter = _sc_kernels()
    degp = sc_degree(dst3d)
    h1p = _tc_mm1(x, W1, degp)
    sp = sc_scatter(h1p, src3d, dst3d)
    h2p = _tc_mid(h1p, sp, degp, W2, b1.reshape(1, HID))
    qp = sc_scatter(h2p, src3d, dst3d)
    return _tc_pool(h2p, qp, degp, b2.reshape(1, OUT), batch3d)
